# baseline (speedup 1.0000x reference)
import jax
import jax.numpy as jnp
from jax.experimental import pallas as pl
from jax.experimental.pallas import tpu as pltpu
from jax.experimental.pallas import tpu_sc as plsc

_NE = 8192
_D = 256
_TB = 256
_KB = 512
_NKB = _NE // _KB
_CCOST = 0.25
_CHUNK_ENDS = (2736, 5472, 8192)


def _segments(j):
    s, e = _KB * j, _KB * (j + 1)
    segs = []
    pos = s
    for ce in _CHUNK_ENDS:
        if pos < ce < e:
            segs.append((pos - s, ce - s, True))
            pos = ce
    segs.append((pos - s, e - s, e in _CHUNK_ENDS))
    return segs


def _tc_body(x_ref, w_ref, x2_ref, w2_ref, idx_ref, cl_ref, cb_ref, tot_ref):
    i = pl.program_id(0)
    n_tok = pl.num_programs(0) * _TB
    x = x_ref[...]
    x2 = x2_ref[...]
    inf = jnp.float32(jnp.inf)

    cur_v = jnp.full((_TB, 1), inf, jnp.float32)
    cur_i = jnp.zeros((_TB, 1), jnp.int32)
    com_b = jnp.full((_TB, 1), inf, jnp.float32)
    com_t = jnp.full((_TB, 1), inf, jnp.float32)
    com_i = jnp.zeros((_TB, 1), jnp.int32)

    for j in range(_NKB):
        w = w_ref[_KB * j:_KB * (j + 1), :]
        m = jax.lax.dot_general(
            x, w, (((1,), (1,)), ((), ())),
            preferred_element_type=jnp.float32)
        w2 = w2_ref[:, _KB * j:_KB * (j + 1)]
        d2 = (x2 + w2) - 2.0 * m
        dist = jnp.sqrt(jnp.maximum(d2, 0.0))
        lanes = jax.lax.broadcasted_iota(jnp.int32, (_TB, _KB), 1)
        for (s, e, commit) in _segments(j):
            if s == 0 and e == _KB:
                dm = dist
            else:
                dm = jnp.where((lanes >= s) & (lanes < e), dist, inf)
            smin = jnp.min(dm, axis=1, keepdims=True)
            sarg = jnp.min(jnp.where(dm == smin, lanes + _KB * j, _NE),
                           axis=1, keepdims=True)
            better = smin < cur_v
            cur_v = jnp.where(better, smin, cur_v)
            cur_i = jnp.where(better, sarg, cur_i)
            if commit:
                take = (cur_v < com_b) | ((cur_v == com_b) & (cur_i < com_i))
                cur_b = cur_v.astype(jnp.bfloat16).astype(jnp.float32)
                com_b = jnp.where(take, cur_b, com_b)
                com_t = jnp.where(take, cur_v, com_t)
                com_i = jnp.where(take, cur_i, com_i)
                cur_v = jnp.full((_TB, 1), inf, jnp.float32)
                cur_i = jnp.zeros((_TB, 1), jnp.int32)

    idx_ref[...] = com_i.reshape(1, 1, _TB)

    mse_part = jnp.sum(com_t * com_t) / (n_tok * _D)

    @pl.when(i == 0)
    def _():
        cl_ref[...] = jnp.zeros((1, 1), jnp.float32)
        cb_ref[...] = jnp.zeros((1, 1), jnp.float32)
        tot_ref[...] = jnp.zeros((1, 1), jnp.float32)

    cl_ref[...] += _CCOST * mse_part
    cb_ref[...] += mse_part
    tot_ref[...] += (1.0 + _CCOST) * mse_part


def _tc_distance_argmin(x_flat, W, x2, w2):
    n_tok = x_flat.shape[0]
    ntb = n_tok // _TB
    scalar = jax.ShapeDtypeStruct((1, 1), jnp.float32)
    return pl.pallas_call(
        _tc_body,
        grid=(ntb,),
        in_specs=[
            pl.BlockSpec((_TB, _D), lambda i: (i, 0)),
            pl.BlockSpec((_NE, _D), lambda i: (0, 0)),
            pl.BlockSpec((_TB, 1), lambda i: (i, 0)),
            pl.BlockSpec((1, _NE), lambda i: (0, 0)),
        ],
        out_specs=[
            pl.BlockSpec((1, 1, _TB), lambda i: (i, 0, 0)),
            pl.BlockSpec((1, 1), lambda i: (0, 0)),
            pl.BlockSpec((1, 1), lambda i: (0, 0)),
            pl.BlockSpec((1, 1), lambda i: (0, 0)),
        ],
        out_shape=[
            jax.ShapeDtypeStruct((ntb, 1, _TB), jnp.int32),
            scalar, scalar, scalar,
        ],
        compiler_params=pltpu.CompilerParams(
            dimension_semantics=("arbitrary",)),
    )(x_flat, W, x2, w2)


_GW = 128


def _sc_gather(W, indices):
    n = indices.shape[0]
    idx2 = indices.reshape(1, n)
    mesh = plsc.VectorSubcoreMesh(core_axis_name="core",
                                  subcore_axis_name="subcore")

    @pl.kernel(out_type=jax.ShapeDtypeStruct((n, _D), W.dtype), mesh=mesh)
    def gather_kernel(w_hbm, i_hbm, o_hbm):
        def body(i_vmem, o_vmem):
            pltpu.sync_copy(w_hbm.at[i_vmem.at[0]], o_vmem)

        pltpu.emit_pipeline(
            body,
            grid=(n // _GW,),
            in_specs=[pl.BlockSpec((1, _GW), lambda i: (0, i))],
            out_specs=[pl.BlockSpec((_GW, _D), lambda i: (i, 0))],
            core_axis_name=("core", "subcore"),
            dimension_semantics=(pltpu.PARALLEL,),
        )(i_hbm, o_hbm)

    return gather_kernel(W, idx2)


def kernel(x, W):
    orig_shape = x.shape
    x_flat = x.reshape(-1, _D)
    x2 = jnp.sum(x_flat * x_flat, axis=1)[:, None]
    w2 = jnp.sum(W * W, axis=1)[None, :]
    idx3, cl, cb, tot = _tc_distance_argmin(x_flat, W, x2, w2)
    indices = idx3.reshape(-1)
    q = _sc_gather(W, indices)
    quantized_st = q.reshape(orig_shape)
    return (quantized_st, indices,
            cl.reshape(()), cb.reshape(()), tot.reshape(()))


# speedup vs baseline: 1.0829x; 1.0829x over previous
import jax
import jax.numpy as jnp
from jax.experimental import pallas as pl
from jax.experimental.pallas import tpu as pltpu
from jax.experimental.pallas import tpu_sc as plsc

_NE = 8192
_D = 256
_TB = 256
_KB = 512
_NKB = _NE // _KB
_CCOST = 0.25
_CHUNK_ENDS = (2736, 5472, 8192)


def _segments(j):
    s, e = _KB * j, _KB * (j + 1)
    segs = []
    pos = s
    for ce in _CHUNK_ENDS:
        if pos < ce < e:
            segs.append((pos - s, ce - s, True))
            pos = ce
    segs.append((pos - s, e - s, e in _CHUNK_ENDS))
    return segs


def _min_tree(v):
    w = v.shape[1]
    while w > 128:
        h = w // 2
        v = jnp.minimum(v[:, :h], v[:, h:w])
        w = h
    return jnp.min(v, axis=1, keepdims=True)


def _tc_body(x_ref, w_ref, x2_ref, w2_ref, idx_ref, cl_ref, cb_ref, tot_ref):
    i = pl.program_id(0)
    n_tok = pl.num_programs(0) * _TB
    x = x_ref[...]
    x2 = x2_ref[...]
    inf = jnp.float32(jnp.inf)

    cur_v = jnp.full((_TB, 1), inf, jnp.float32)
    cur_i = jnp.zeros((_TB, 1), jnp.float32)
    com_b = jnp.full((_TB, 1), inf, jnp.float32)
    com_t = jnp.full((_TB, 1), inf, jnp.float32)
    com_i = jnp.zeros((_TB, 1), jnp.float32)

    lanes = jax.lax.broadcasted_iota(
        jnp.int32, (_TB, _KB), 1).astype(jnp.float32)
    for j in range(_NKB):
        w = w_ref[_KB * j:_KB * (j + 1), :]
        m = jax.lax.dot_general(
            x, w, (((1,), (1,)), ((), ())),
            preferred_element_type=jnp.float32)
        w2 = w2_ref[:, _KB * j:_KB * (j + 1)]
        d2 = (x2 + w2) - 2.0 * m
        dist = jnp.sqrt(jnp.maximum(d2, 0.0))
        for (s, e, commit) in _segments(j):
            if s == 0 and e == _KB:
                dm = dist
            else:
                dm = jnp.where((lanes >= float(s)) & (lanes < float(e)),
                               dist, inf)
            smin = _min_tree(dm)
            sarg = _min_tree(jnp.where(dm == smin, lanes, float(_NE)))
            sarg = sarg + float(_KB * j)
            better = smin < cur_v
            cur_v = jnp.where(better, smin, cur_v)
            cur_i = jnp.where(better, sarg, cur_i)
            if commit:
                take = (cur_v < com_b) | ((cur_v == com_b) & (cur_i < com_i))
                cur_b = cur_v.astype(jnp.bfloat16).astype(jnp.float32)
                com_b = jnp.where(take, cur_b, com_b)
                com_t = jnp.where(take, cur_v, com_t)
                com_i = jnp.where(take, cur_i, com_i)
                cur_v = jnp.full((_TB, 1), inf, jnp.float32)
                cur_i = jnp.zeros((_TB, 1), jnp.float32)

    idx_ref[...] = com_i.astype(jnp.int32).reshape(1, 1, _TB)

    mse_part = jnp.sum(com_t * com_t) / (n_tok * _D)

    @pl.when(i == 0)
    def _():
        cl_ref[...] = jnp.zeros((1, 1), jnp.float32)
        cb_ref[...] = jnp.zeros((1, 1), jnp.float32)
        tot_ref[...] = jnp.zeros((1, 1), jnp.float32)

    cl_ref[...] += _CCOST * mse_part
    cb_ref[...] += mse_part
    tot_ref[...] += (1.0 + _CCOST) * mse_part


def _tc_distance_argmin(x_flat, W, x2, w2):
    n_tok = x_flat.shape[0]
    ntb = n_tok // _TB
    scalar = jax.ShapeDtypeStruct((1, 1), jnp.float32)
    return pl.pallas_call(
        _tc_body,
        grid=(ntb,),
        in_specs=[
            pl.BlockSpec((_TB, _D), lambda i: (i, 0)),
            pl.BlockSpec((_NE, _D), lambda i: (0, 0)),
            pl.BlockSpec((_TB, 1), lambda i: (i, 0)),
            pl.BlockSpec((1, _NE), lambda i: (0, 0)),
        ],
        out_specs=[
            pl.BlockSpec((1, 1, _TB), lambda i: (i, 0, 0)),
            pl.BlockSpec((1, 1), lambda i: (0, 0)),
            pl.BlockSpec((1, 1), lambda i: (0, 0)),
            pl.BlockSpec((1, 1), lambda i: (0, 0)),
        ],
        out_shape=[
            jax.ShapeDtypeStruct((ntb, 1, _TB), jnp.int32),
            scalar, scalar, scalar,
        ],
        compiler_params=pltpu.CompilerParams(
            dimension_semantics=("arbitrary",)),
    )(x_flat, W, x2, w2)


_GW = 128


def _sc_gather(W, indices):
    n = indices.shape[0]
    idx2 = indices.reshape(1, n)
    mesh = plsc.VectorSubcoreMesh(core_axis_name="core",
                                  subcore_axis_name="subcore")

    @pl.kernel(out_type=jax.ShapeDtypeStruct((n, _D), W.dtype), mesh=mesh)
    def gather_kernel(w_hbm, i_hbm, o_hbm):
        def body(i_vmem, o_vmem):
            pltpu.sync_copy(w_hbm.at[i_vmem.at[0]], o_vmem)

        pltpu.emit_pipeline(
            body,
            grid=(n // _GW,),
            in_specs=[pl.BlockSpec((1, _GW), lambda i: (0, i))],
            out_specs=[pl.BlockSpec((_GW, _D), lambda i: (i, 0))],
            core_axis_name=("core", "subcore"),
            dimension_semantics=(pltpu.PARALLEL,),
        )(i_hbm, o_hbm)

    return gather_kernel(W, idx2)


def kernel(x, W):
    orig_shape = x.shape
    x_flat = x.reshape(-1, _D)
    x2 = jnp.sum(x_flat * x_flat, axis=1)[:, None]
    w2 = jnp.sum(W * W, axis=1)[None, :]
    idx3, cl, cb, tot = _tc_distance_argmin(x_flat, W, x2, w2)
    indices = idx3.reshape(-1)
    q = _sc_gather(W, indices)
    quantized_st = q.reshape(orig_shape)
    return (quantized_st, indices,
            cl.reshape(()), cb.reshape(()), tot.reshape(()))


# speedup vs baseline: 1.2067x; 1.1143x over previous
import jax
import jax.numpy as jnp
from jax.experimental import pallas as pl
from jax.experimental.pallas import tpu as pltpu
from jax.experimental.pallas import tpu_sc as plsc

_NE = 8192
_D = 256
_TB = 256
_KB = 512
_NKB = _NE // _KB
_CCOST = 0.25
_CHUNK_ENDS = (2736, 5472, 8192)


def _pieces(j):
    s, e = _KB * j, _KB * (j + 1)
    bounds = [b for b in _CHUNK_ENDS if s < b < e]
    out = []
    pos = s
    for b in bounds + [e]:
        cid = sum(1 for ce in _CHUNK_ENDS if ce <= pos)
        out.append((pos - s, b - s, cid))
        pos = b
    return out


def _fold128(v):
    v = jnp.minimum(v[:, :256], v[:, 256:])
    return jnp.minimum(v[:, :128], v[:, 128:])


def _tc_body(x_ref, w_ref, x2_ref, w2_ref,
             idx_ref, cl_ref, cb_ref, tot_ref, d2s_ref):
    i = pl.program_id(0)
    n_tok = pl.num_programs(0) * _TB
    x = x_ref[...]
    x2 = x2_ref[...]
    x2x = x + x
    inf = jnp.float32(jnp.inf)
    lanes = jax.lax.broadcasted_iota(jnp.int32, (_TB, _KB), 1)

    folds = [jnp.full((_TB, 128), inf, jnp.float32) for _ in range(3)]
    for j in range(_NKB):
        w = w_ref[_KB * j:_KB * (j + 1), :]
        m2 = jax.lax.dot_general(
            x2x, w, (((1,), (1,)), ((), ())),
            preferred_element_type=jnp.float32)
        w2 = w2_ref[:, _KB * j:_KB * (j + 1)]
        d2 = (x2 + w2) - m2
        d2s_ref[:, _KB * j:_KB * (j + 1)] = d2
        for (s, e, c) in _pieces(j):
            if s == 0 and e == _KB:
                v = d2
            else:
                v = jnp.where((lanes >= s) & (lanes < e), d2, inf)
            folds[c] = jnp.minimum(folds[c], _fold128(v))

    com_b = jnp.full((_TB, 1), inf, jnp.float32)
    com_t = jnp.full((_TB, 1), inf, jnp.float32)
    com_i = jnp.zeros((_TB, 1), jnp.int32)
    for c in range(3):
        d2min = jnp.min(folds[c], axis=1, keepdims=True)
        sv = jnp.sqrt(jnp.maximum(d2min, 0.0))
        nxt = jax.lax.bitcast_convert_type(
            jax.lax.bitcast_convert_type(sv, jnp.int32) + 1, jnp.float32)
        u = nxt - sv
        p = sv * sv
        cd = sv * 4097.0
        hi = cd - (cd - sv)
        lo = sv - hi
        err = ((hi * hi - p) + 2.0 * (hi * lo)) + lo * lo
        lo2 = (err + sv * u) + 0.25 * (u * u)
        acc = jnp.full((_TB, 128), 4 * _NE, jnp.int32)
        for j in range(_NKB):
            pieces = [pp for pp in _pieces(j) if pp[2] == c]
            if not pieces:
                continue
            d2b = d2s_ref[:, _KB * j:_KB * (j + 1)]
            tie = (d2b - p) <= lo2
            (s, e, _) = pieces[0]
            if not (s == 0 and e == _KB):
                tie = tie & (lanes >= s) & (lanes < e)
            cand = jnp.where(tie, lanes, _NE)
            acc = jnp.minimum(acc, _fold128(cand) + _KB * j)
        idxc = jnp.min(acc, axis=1, keepdims=True)
        take = (sv < com_b) | ((sv == com_b) & (idxc < com_i))
        cur_b = sv.astype(jnp.bfloat16).astype(jnp.float32)
        com_b = jnp.where(take, cur_b, com_b)
        com_t = jnp.where(take, sv, com_t)
        com_i = jnp.where(take, idxc, com_i)

    idx_ref[...] = com_i.reshape(1, 1, _TB)

    mse_part = jnp.sum(com_t * com_t) / (n_tok * _D)

    @pl.when(i == 0)
    def _():
        cl_ref[...] = jnp.zeros((1, 1), jnp.float32)
        cb_ref[...] = jnp.zeros((1, 1), jnp.float32)
        tot_ref[...] = jnp.zeros((1, 1), jnp.float32)

    cl_ref[...] += _CCOST * mse_part
    cb_ref[...] += mse_part
    tot_ref[...] += (1.0 + _CCOST) * mse_part


def _tc_distance_argmin(x_flat, W, x2, w2):
    n_tok = x_flat.shape[0]
    ntb = n_tok // _TB
    scalar = jax.ShapeDtypeStruct((1, 1), jnp.float32)
    return pl.pallas_call(
        _tc_body,
        grid=(ntb,),
        in_specs=[
            pl.BlockSpec((_TB, _D), lambda i: (i, 0)),
            pl.BlockSpec((_NE, _D), lambda i: (0, 0)),
            pl.BlockSpec((_TB, 1), lambda i: (i, 0)),
            pl.BlockSpec((1, _NE), lambda i: (0, 0)),
        ],
        out_specs=[
            pl.BlockSpec((1, 1, _TB), lambda i: (i, 0, 0)),
            pl.BlockSpec((1, 1), lambda i: (0, 0)),
            pl.BlockSpec((1, 1), lambda i: (0, 0)),
            pl.BlockSpec((1, 1), lambda i: (0, 0)),
        ],
        out_shape=[
            jax.ShapeDtypeStruct((ntb, 1, _TB), jnp.int32),
            scalar, scalar, scalar,
        ],
        scratch_shapes=[pltpu.VMEM((_TB, _NE), jnp.float32)],
        compiler_params=pltpu.CompilerParams(
            dimension_semantics=("arbitrary",)),
    )(x_flat, W, x2, w2)


_GW = 128


def _sc_gather(W, indices):
    n = indices.shape[0]
    idx2 = indices.reshape(1, n)
    mesh = plsc.VectorSubcoreMesh(core_axis_name="core",
                                  subcore_axis_name="subcore")

    @pl.kernel(out_type=jax.ShapeDtypeStruct((n, _D), W.dtype), mesh=mesh)
    def gather_kernel(w_hbm, i_hbm, o_hbm):
        def body(i_vmem, o_vmem):
            pltpu.sync_copy(w_hbm.at[i_vmem.at[0]], o_vmem)

        pltpu.emit_pipeline(
            body,
            grid=(n // _GW,),
            in_specs=[pl.BlockSpec((1, _GW), lambda i: (0, i))],
            out_specs=[pl.BlockSpec((_GW, _D), lambda i: (i, 0))],
            core_axis_name=("core", "subcore"),
            dimension_semantics=(pltpu.PARALLEL,),
        )(i_hbm, o_hbm)

    return gather_kernel(W, idx2)


def kernel(x, W):
    orig_shape = x.shape
    x_flat = x.reshape(-1, _D)
    x2 = jnp.sum(x_flat * x_flat, axis=1)[:, None]
    w2 = jnp.sum(W * W, axis=1)[None, :]
    idx3, cl, cb, tot = _tc_distance_argmin(x_flat, W, x2, w2)
    indices = idx3.reshape(-1)
    q = _sc_gather(W, indices)
    quantized_st = q.reshape(orig_shape)
    return (quantized_st, indices,
            cl.reshape(()), cb.reshape(()), tot.reshape(()))
